# speedup vs baseline: 1.0036x; 1.0036x over previous
import functools

import jax
import jax.numpy as jnp
from jax import lax
from jax.experimental import pallas as pl
from jax.experimental.pallas import tpu as pltpu
from jax.experimental.pallas import tpu_sc as plsc

_CHUNK = 128
_NBUF = 4


def _lookup(idx2, table):
    n_rows, chunk = idx2.shape
    v, d = table.shape
    info = plsc.get_sparse_core_info()
    nw = info.num_cores * info.num_subcores
    n_chunks = n_rows // nw
    b_per_w = n_chunks * chunk
    nbuf = min(_NBUF, n_chunks)

    mesh = plsc.VectorSubcoreMesh(core_axis_name="c", subcore_axis_name="s")

    @functools.partial(
        pl.kernel,
        mesh=mesh,
        out_type=jax.ShapeDtypeStruct((n_rows * chunk, d), jnp.float32),
        scratch_types=(
            [pltpu.VMEM((n_chunks, chunk), jnp.int32)]
            + [pltpu.VMEM((chunk, d), jnp.float32) for _ in range(nbuf)]
            + [pltpu.SemaphoreType.DMA for _ in range(2 * nbuf)]
        ),
    )
    def k(table_hbm, idx_hbm, out_hbm, idx_v, *rest):
        bufs = rest[:nbuf]
        sems_g = rest[nbuf : 2 * nbuf]
        sems_s = rest[2 * nbuf : 3 * nbuf]
        wid = lax.axis_index("s") * info.num_cores + lax.axis_index("c")
        pltpu.sync_copy(idx_hbm.at[pl.ds(wid * n_chunks, n_chunks)], idx_v)

        gath = {}
        scat = {}

        def start_gather(c):
            b = c % nbuf
            gath[c] = pltpu.async_copy(
                table_hbm.at[idx_v.at[c]], bufs[b], sems_g[b]
            )

        for c in range(nbuf):
            start_gather(c)
        out_base = wid * b_per_w
        for c in range(n_chunks):
            b = c % nbuf
            gath[c].wait()
            scat[c] = pltpu.async_copy(
                bufs[b], out_hbm.at[pl.ds(out_base + c * chunk, chunk)], sems_s[b]
            )
            nxt = c + nbuf
            if nxt < n_chunks:
                scat[c].wait()
                start_gather(nxt)
        for c in range(n_chunks - nbuf, n_chunks):
            scat[c].wait()

    return k(table, idx2)


def kernel(input, table):
    d = table.shape[-1]
    idx = input.reshape(-1).astype(jnp.int32)
    idx2 = idx.reshape(-1, _CHUNK)
    out = _lookup(idx2, table.astype(jnp.float32))
    return out.reshape(input.shape + (d,))


# speedup vs baseline: 4.4366x; 4.4208x over previous
import functools

import jax
import jax.numpy as jnp
from jax import lax
from jax.experimental import pallas as pl
from jax.experimental.pallas import tpu as pltpu
from jax.experimental.pallas import tpu_sc as plsc

_CHUNK = 128
_NBUF = 4


def _lookup(idx2, table):
    n_rows, chunk = idx2.shape
    v, d = table.shape
    info = plsc.get_sparse_core_info()
    nw = info.num_cores * info.num_subcores
    n_chunks = n_rows // nw
    b_per_w = n_chunks * chunk
    nbuf = min(_NBUF, n_chunks)

    mesh = plsc.VectorSubcoreMesh(core_axis_name="c", subcore_axis_name="s")

    @functools.partial(
        pl.kernel,
        mesh=mesh,
        out_type=jax.ShapeDtypeStruct((n_rows * chunk, d), jnp.float32),
        scratch_types=(
            [pltpu.VMEM_SHARED((v, d), jnp.float32)]
            + [pltpu.VMEM((n_chunks, chunk), jnp.int32)]
            + [pltpu.VMEM((chunk, d), jnp.float32) for _ in range(nbuf)]
            + [pltpu.SemaphoreType.DMA for _ in range(2 * nbuf)]
        ),
    )
    def k(table_hbm, idx_hbm, out_hbm, table_sh, idx_v, *rest):
        bufs = rest[:nbuf]
        sems_g = rest[nbuf : 2 * nbuf]
        sems_s = rest[2 * nbuf : 3 * nbuf]
        sid = lax.axis_index("s")
        wid = sid * info.num_cores + lax.axis_index("c")
        @pl.when(sid == 0)
        def _():
            pltpu.sync_copy(table_hbm, table_sh)

        pltpu.sync_copy(idx_hbm.at[pl.ds(wid * n_chunks, n_chunks)], idx_v)
        plsc.subcore_barrier()

        gath = {}
        scat = {}

        def start_gather(c):
            b = c % nbuf
            gath[c] = pltpu.async_copy(
                table_sh.at[idx_v.at[c]], bufs[b], sems_g[b]
            )

        for c in range(nbuf):
            start_gather(c)
        out_base = wid * b_per_w
        for c in range(n_chunks):
            b = c % nbuf
            gath[c].wait()
            scat[c] = pltpu.async_copy(
                bufs[b], out_hbm.at[pl.ds(out_base + c * chunk, chunk)], sems_s[b]
            )
            nxt = c + nbuf
            if nxt < n_chunks:
                scat[c].wait()
                start_gather(nxt)
        for c in range(n_chunks - nbuf, n_chunks):
            scat[c].wait()

    return k(table, idx2)


def kernel(input, table):
    d = table.shape[-1]
    idx = input.reshape(-1).astype(jnp.int32)
    idx2 = idx.reshape(-1, _CHUNK)
    out = _lookup(idx2, table.astype(jnp.float32))
    return out.reshape(input.shape + (d,))
